# baseline (speedup 1.0000x reference)
import functools

import jax
import jax.numpy as jnp
from jax import lax
from jax.experimental import pallas as pl
from jax.experimental.pallas import tpu as pltpu
from jax.experimental.pallas import tpu_sc as plsc

BATCH, SEQ, EMBED = 4096, 50, 128
NC, NS = 2, 16
NW = NC * NS
ROWS_PER_W = BATCH // NW
KB = 2
CHUNK = KB * SEQ
NCHUNK = ROWS_PER_W // KB
NBUF = 8
NGROUP = NCHUNK // NBUF


def _sc_body(idx_hbm, table_hbm, out_hbm, idx_v, rows_v, gsem):
    wid = lax.axis_index("s") * NC + lax.axis_index("c")
    base = wid * ROWS_PER_W

    pltpu.sync_copy(idx_hbm.at[wid], idx_v)

    def issue(j, b):
        pltpu.async_copy(table_hbm.at[idx_v.at[j]], rows_v.at[b], gsem.at[b])

    def drain(j, b):
        pltpu.make_async_copy(
            table_hbm.at[idx_v.at[0]], rows_v.at[b], gsem.at[b]
        ).wait()
        bb = base + j * KB
        for r in range(KB):
            pltpu.sync_copy(
                rows_v.at[b].at[pl.ds(r * SEQ, SEQ)], out_hbm.at[bb + r]
            )

    for b in range(NBUF):
        issue(b, b)

    def group(g, carry):
        for b in range(NBUF):
            j = g * NBUF + b
            drain(j, b)
            issue(j + NBUF, b)
        return carry

    lax.fori_loop(0, NGROUP - 1, group, 0)

    for b in range(NBUF):
        drain((NGROUP - 1) * NBUF + b, b)


def _sc_gather(idx, table):
    mesh = plsc.VectorSubcoreMesh(
        core_axis_name="c", subcore_axis_name="s", num_cores=NC, num_subcores=NS
    )
    run = functools.partial(
        pl.kernel,
        out_type=jax.ShapeDtypeStruct((BATCH, SEQ, EMBED), jnp.float32),
        mesh=mesh,
        scratch_types=[
            pltpu.VMEM((NCHUNK, CHUNK), jnp.int32),
            pltpu.VMEM((NBUF, CHUNK, EMBED), jnp.float32),
            pltpu.SemaphoreType.DMA((NBUF,)),
        ],
    )(_sc_body)
    return run(idx, table)


@jax.jit
def kernel(x, embedding):
    idx = x.astype(jnp.int32).reshape(NW, NCHUNK, CHUNK)
    return _sc_gather(idx, embedding)
